# baseline (speedup 1.0000x reference)
import jax
import jax.numpy as jnp
from jax.experimental import pallas as pl
from jax.experimental.pallas import tpu as pltpu

D = 2048
H = 256
E = 8
K = 2
B = 2
S = 2048

TM = 512
TC = 256


def _gate_kernel(x_ref, w_ref, b_ref, gated_ref, psum_ref):
    i = pl.program_id(0)
    xt = x_ref[...]
    z = jnp.dot(xt.astype(jnp.bfloat16), w_ref[...],
                preferred_element_type=jnp.float32)
    g = xt * jax.nn.sigmoid(z + b_ref[...])
    gated_ref[...] = g
    colsum = jnp.sum(g, axis=0, keepdims=True)[None]

    @pl.when(i % (S // TM) == 0)
    def _init():
        psum_ref[...] = colsum

    @pl.when(i % (S // TM) != 0)
    def _acc():
        psum_ref[...] += colsum


def _router_kernel(psum_ref, w1_ref, b1_ref, w2_ref, b2_ref,
                   probs_ref, gains_ref):
    pooled = psum_ref[...].reshape(B, D) * (1.0 / S)
    h = jnp.tanh(
        jnp.dot(pooled.astype(jnp.bfloat16), w1_ref[...],
                preferred_element_type=jnp.float32) + b1_ref[...])
    logits = (jnp.dot(h.astype(jnp.bfloat16), w2_ref[...],
                      preferred_element_type=jnp.float32) + b2_ref[...])
    ids = jax.lax.broadcasted_iota(jnp.int32, (B, E), 1)
    v1 = jnp.max(logits, axis=1, keepdims=True)
    i1 = jnp.min(jnp.where(logits == v1, ids, E), axis=1, keepdims=True)
    m1 = ids == i1
    masked = jnp.where(m1, -jnp.inf, logits)
    v2 = jnp.max(masked, axis=1, keepdims=True)
    i2 = jnp.min(jnp.where(masked == v2, ids, E), axis=1, keepdims=True)
    m2 = ids == i2
    ex = jnp.exp(logits - v1)
    probs_ref[...] = ex / jnp.sum(ex, axis=1, keepdims=True)
    e2 = jnp.exp(v2 - v1)
    w1 = 1.0 / (1.0 + e2)
    w2 = e2 * w1
    gains_ref[...] = jnp.where(m1, w1, 0.0) + jnp.where(m2, w2, 0.0)


def _bcast_kernel(gains_ref, gated_ref, out_ref):
    e = pl.program_id(2)
    g = gated_ref[0]
    gv = gains_ref[0]
    ids = jax.lax.broadcasted_iota(jnp.int32, (1, E), 1)
    ge = jnp.sum(jnp.where(ids == e, gv, 0.0))
    out_ref[0, 0] = g * ge


def kernel(x, gate_W, gate_b, W1, b1, W2, b2):
    xf = x.reshape(B * S, D)
    wb = gate_W.astype(jnp.bfloat16)

    gated, psum = pl.pallas_call(
        _gate_kernel,
        grid=(B * S // TM,),
        in_specs=[
            pl.BlockSpec((TM, D), lambda i: (i, 0)),
            pl.BlockSpec((D, D), lambda i: (0, 0)),
            pl.BlockSpec((1, D), lambda i: (0, 0)),
        ],
        out_specs=[
            pl.BlockSpec((TM, D), lambda i: (i, 0)),
            pl.BlockSpec((1, 1, D), lambda i: (i // (S // TM), 0, 0)),
        ],
        out_shape=[
            jax.ShapeDtypeStruct((B * S, D), jnp.float32),
            jax.ShapeDtypeStruct((B, 1, D), jnp.float32),
        ],
        compiler_params=pltpu.CompilerParams(
            dimension_semantics=("arbitrary",)),
    )(xf, wb, gate_b.reshape(1, D))

    probs, gains = pl.pallas_call(
        _router_kernel,
        out_shape=[
            jax.ShapeDtypeStruct((B, E), jnp.float32),
            jax.ShapeDtypeStruct((B, E), jnp.float32),
        ],
    )(psum.reshape(B, D), W1.astype(jnp.bfloat16), b1.reshape(1, H),
      W2.astype(jnp.bfloat16), b2.reshape(1, E))

    routed = pl.pallas_call(
        _bcast_kernel,
        grid=(B, S // TC, E),
        in_specs=[
            pl.BlockSpec((1, 1, E), lambda b, i, e: (b, 0, 0)),
            pl.BlockSpec((1, TC, D), lambda b, i, e: (b, i, 0)),
        ],
        out_specs=pl.BlockSpec((1, 1, TC, D), lambda b, i, e: (e, b, i, 0)),
        out_shape=jax.ShapeDtypeStruct((E, B, S, D), jnp.float32),
        compiler_params=pltpu.CompilerParams(
            dimension_semantics=("parallel", "arbitrary", "arbitrary")),
    )(gains.reshape(B, 1, E), gated.reshape(B, S, D))

    return routed, probs


# speedup vs baseline: 1.2550x; 1.2550x over previous
import jax
import jax.numpy as jnp
from jax.experimental import pallas as pl
from jax.experimental.pallas import tpu as pltpu

D = 2048
H = 256
E = 8
K = 2
B = 2
S = 2048

TM = 512
TC = 256


def _gate_kernel(x_ref, w_ref, b_ref, gated_ref, psum_ref):
    i = pl.program_id(0)
    xt = x_ref[...]
    z = jnp.dot(xt.astype(jnp.bfloat16), w_ref[...],
                preferred_element_type=jnp.float32)
    g = xt * jax.nn.sigmoid(z + b_ref[...])
    gated_ref[...] = g
    colsum = jnp.sum(g, axis=0, keepdims=True)[None]

    @pl.when(i % (S // TM) == 0)
    def _init():
        psum_ref[...] = colsum

    @pl.when(i % (S // TM) != 0)
    def _acc():
        psum_ref[...] += colsum


def _router_kernel(psum_ref, w1_ref, b1_ref, w2_ref, b2_ref,
                   probs_ref, gains_ref):
    pooled = psum_ref[...].reshape(B, D) * (1.0 / S)
    h = jnp.tanh(
        jnp.dot(pooled.astype(jnp.bfloat16), w1_ref[...],
                preferred_element_type=jnp.float32) + b1_ref[...])
    logits = (jnp.dot(h.astype(jnp.bfloat16), w2_ref[...],
                      preferred_element_type=jnp.float32) + b2_ref[...])
    ids = jax.lax.broadcasted_iota(jnp.int32, (B, E), 1)
    v1 = jnp.max(logits, axis=1, keepdims=True)
    i1 = jnp.min(jnp.where(logits == v1, ids, E), axis=1, keepdims=True)
    m1 = ids == i1
    masked = jnp.where(m1, -jnp.inf, logits)
    v2 = jnp.max(masked, axis=1, keepdims=True)
    i2 = jnp.min(jnp.where(masked == v2, ids, E), axis=1, keepdims=True)
    m2 = ids == i2
    ex = jnp.exp(logits - v1)
    probs_ref[...] = ex / jnp.sum(ex, axis=1, keepdims=True)
    e2 = jnp.exp(v2 - v1)
    w1 = 1.0 / (1.0 + e2)
    w2 = e2 * w1
    gains_ref[...] = jnp.where(m1, w1, 0.0) + jnp.where(m2, w2, 0.0)


def _bcast_kernel(gains_ref, gated_ref, out_ref):
    g = gated_ref[0]
    gv = gains_ref[0]
    for e in range(E):
        out_ref[e, 0] = g * gv[0, e]


def kernel(x, gate_W, gate_b, W1, b1, W2, b2):
    xf = x.reshape(B * S, D)
    wb = gate_W.astype(jnp.bfloat16)

    gated, psum = pl.pallas_call(
        _gate_kernel,
        grid=(B * S // TM,),
        in_specs=[
            pl.BlockSpec((TM, D), lambda i: (i, 0)),
            pl.BlockSpec((D, D), lambda i: (0, 0)),
            pl.BlockSpec((1, D), lambda i: (0, 0)),
        ],
        out_specs=[
            pl.BlockSpec((TM, D), lambda i: (i, 0)),
            pl.BlockSpec((1, 1, D), lambda i: (i // (S // TM), 0, 0)),
        ],
        out_shape=[
            jax.ShapeDtypeStruct((B * S, D), jnp.float32),
            jax.ShapeDtypeStruct((B, 1, D), jnp.float32),
        ],
        compiler_params=pltpu.CompilerParams(
            dimension_semantics=("arbitrary",)),
    )(xf, wb, gate_b.reshape(1, D))

    probs, gains = pl.pallas_call(
        _router_kernel,
        out_shape=[
            jax.ShapeDtypeStruct((B, E), jnp.float32),
            jax.ShapeDtypeStruct((B, E), jnp.float32),
        ],
    )(psum.reshape(B, D), W1.astype(jnp.bfloat16), b1.reshape(1, H),
      W2.astype(jnp.bfloat16), b2.reshape(1, E))

    routed = pl.pallas_call(
        _bcast_kernel,
        grid=(B, S // TC),
        in_specs=[
            pl.BlockSpec((1, 1, E), lambda b, i: (b, 0, 0)),
            pl.BlockSpec((1, TC, D), lambda b, i: (b, i, 0)),
        ],
        out_specs=pl.BlockSpec((E, 1, TC, D), lambda b, i: (0, b, i, 0)),
        out_shape=jax.ShapeDtypeStruct((E, B, S, D), jnp.float32),
        compiler_params=pltpu.CompilerParams(
            dimension_semantics=("parallel", "parallel")),
    )(gains.reshape(B, 1, E), gated.reshape(B, S, D))

    return routed, probs


# speedup vs baseline: 1.4816x; 1.1806x over previous
import jax
import jax.numpy as jnp
from jax.experimental import pallas as pl
from jax.experimental.pallas import tpu as pltpu

D = 2048
H = 256
E = 8
B = 2
S = 2048

TM = 256
TC = 64
PH1 = B * S // TM
P = S // TM
NSB = S // TC
NOUT = B * NSB


def _mega_kernel(x_ref, w_ref, gb_ref, w1_ref, b1_ref, w2_ref, b2_ref,
                 routed_ref, probs_ref,
                 wb_s, gated_s, psum_s, gains_s):
    i = pl.program_id(0)

    @pl.when(i == 0)
    def _cast_w():
        wb_s[...] = w_ref[...].astype(jnp.bfloat16)

    @pl.when(i < PH1)
    def _mm():
        xt = x_ref[...]
        z = jnp.dot(xt.astype(jnp.bfloat16), wb_s[...],
                    preferred_element_type=jnp.float32) + gb_ref[...]
        g = xt * jax.nn.sigmoid(z)
        gated_s[pl.ds(i * TM, TM), :] = g.astype(jnp.bfloat16)
        colsum = jnp.sum(g, axis=0, keepdims=True)

        for r in range(PH1):
            if r in (P - 1, 2 * P - 1):
                continue
            @pl.when(i == r)
            def _store(r=r):
                psum_s[r:r + 1, :] = colsum

        for bb, last in ((0, P - 1), (1, 2 * P - 1)):
            @pl.when(i == last)
            def _router(bb=bb, last=last):
                ps = psum_s[...]
                prev = jnp.sum(ps[last - (P - 1):last, :], axis=0,
                               keepdims=True)
                pooled = (prev + colsum) * (1.0 / S)
                h = jnp.tanh(
                    jnp.dot(pooled.astype(jnp.bfloat16), w1_ref[...],
                            preferred_element_type=jnp.float32) + b1_ref[...])
                logits = (jnp.dot(h.astype(jnp.bfloat16),
                                  w2_ref[...].astype(jnp.bfloat16),
                                  preferred_element_type=jnp.float32)
                          + b2_ref[...])
                ids = jax.lax.broadcasted_iota(jnp.int32, (1, E), 1)
                v1 = jnp.max(logits, axis=1, keepdims=True)
                i1 = jnp.min(jnp.where(logits == v1, ids, E),
                             axis=1, keepdims=True)
                m1 = ids == i1
                masked = jnp.where(m1, -jnp.inf, logits)
                v2 = jnp.max(masked, axis=1, keepdims=True)
                i2 = jnp.min(jnp.where(masked == v2, ids, E),
                             axis=1, keepdims=True)
                m2 = ids == i2
                ex = jnp.exp(logits - v1)
                probs_ref[bb:bb + 1, :] = ex / jnp.sum(ex, axis=1,
                                                       keepdims=True)
                e2 = jnp.exp(v2 - v1)
                wt1 = 1.0 / (1.0 + e2)
                wt2 = e2 * wt1
                gains_s[bb:bb + 1, :] = (jnp.where(m1, wt1, 0.0)
                                         + jnp.where(m2, wt2, 0.0))

    @pl.when(i >= P)
    def _out():
        jj = i - P
        bb = jj // NSB
        g = gated_s[pl.ds(jj * TC, TC), :].astype(jnp.float32)
        gv = gains_s[...]
        rid = jax.lax.broadcasted_iota(jnp.int32, (B, E), 0)
        row = jnp.sum(jnp.where(rid == bb, gv, 0.0), axis=0,
                      keepdims=True)
        for e in range(E):
            routed_ref[e, 0] = g * row[0, e]


def kernel(x, gate_W, gate_b, W1, b1, W2, b2):
    xf = x.reshape(B * S, D)

    routed, probs = pl.pallas_call(
        _mega_kernel,
        grid=(P + NOUT,),
        in_specs=[
            pl.BlockSpec((TM, D), lambda i: (jnp.minimum(i, PH1 - 1), 0)),
            pl.BlockSpec((D, D), lambda i: (0, 0)),
            pl.BlockSpec((1, D), lambda i: (0, 0)),
            pl.BlockSpec((D, H), lambda i: (0, 0)),
            pl.BlockSpec((1, H), lambda i: (0, 0)),
            pl.BlockSpec((H, E), lambda i: (0, 0)),
            pl.BlockSpec((1, E), lambda i: (0, 0)),
        ],
        out_specs=[
            pl.BlockSpec(
                (E, 1, TC, D),
                lambda i: (0, jnp.maximum(i - P, 0) // NSB,
                           jnp.maximum(i - P, 0) % NSB, 0)),
            pl.BlockSpec((B, E), lambda i: (0, 0)),
        ],
        out_shape=[
            jax.ShapeDtypeStruct((E, B, S, D), jnp.float32),
            jax.ShapeDtypeStruct((B, E), jnp.float32),
        ],
        scratch_shapes=[
            pltpu.VMEM((D, D), jnp.bfloat16),
            pltpu.VMEM((B * S, D), jnp.bfloat16),
            pltpu.VMEM((PH1, D), jnp.float32),
            pltpu.VMEM((B, E), jnp.float32),
        ],
        compiler_params=pltpu.CompilerParams(
            dimension_semantics=("arbitrary",)),
    )(xf, gate_W, gate_b.reshape(1, D), W1.astype(jnp.bfloat16),
      b1.reshape(1, H), W2, b2.reshape(1, E))

    return routed, probs
